# Initial kernel scaffold; baseline (speedup 1.0000x reference)
import functools

import jax
import jax.numpy as jnp
from jax import lax
from jax.experimental import pallas as pl
from jax.experimental.pallas import tpu as pltpu
from jax.experimental.pallas import tpu_sc as plsc

NN = 100000
EE = 1600000
NP = 102400
NSUB = 16
NCORE = 2
ROWS_PER_SUB = NP // NSUB
RB = 12800


def _fill_f32(ref, n, value):
  def body(i, _):
    ref[pl.ds(i * 16, 16)] = jnp.full((16,), value, jnp.float32)
    return 0
  lax.fori_loop(0, n // 16, body, 0)


def _fill_rows_f32(ref, rows, value):
  def body(i, _):
    ref[i, :] = jnp.full((16,), value, jnp.float32)
    return 0
  lax.fori_loop(0, rows, body, 0)


def _sc_degree(dst):
  B = 10000
  epw = EE // (NCORE * NSUB)
  nblk = epw // B
  mesh = plsc.VectorSubcoreMesh(core_axis_name="c", subcore_axis_name="s")

  @functools.partial(
      pl.kernel,
      out_type=jax.ShapeDtypeStruct((NCORE, NP), jnp.float32),
      mesh=mesh,
      scratch_types=[
          pltpu.VMEM((B,), jnp.int32),
          pltpu.VMEM((B,), jnp.float32),
          pltpu.VMEM((ROWS_PER_SUB,), jnp.float32),
      ],
  )
  def deg_kernel(dst_hbm, out_hbm, idx_v, ones_v, zero_v):
    c = lax.axis_index("c")
    s = lax.axis_index("s")
    wid = c * NSUB + s
    _fill_f32(ones_v, B, 1.0)
    _fill_f32(zero_v, ROWS_PER_SUB, 0.0)

    def run(acc):
      pltpu.sync_copy(zero_v, acc.at[pl.ds(s * ROWS_PER_SUB, ROWS_PER_SUB)])
      plsc.subcore_barrier()
      base = wid * epw

      def body(i, _):
        pltpu.sync_copy(dst_hbm.at[pl.ds(base + i * B, B)], idx_v)
        pltpu.sync_copy(ones_v, acc.at[idx_v], add=True)
        return 0

      lax.fori_loop(0, nblk, body, 0)
      plsc.subcore_barrier()
      sl = pl.ds(s * ROWS_PER_SUB, ROWS_PER_SUB)
      pltpu.sync_copy(acc.at[sl], out_hbm.at[c].at[sl])

    pl.run_scoped(run, pltpu.VMEM_SHARED((NP,), jnp.float32))

  return deg_kernel(dst)


def _sc_agg16_edge_split(src, dst, z):
  B = 5000
  epw = EE // (NCORE * NSUB)
  nblk = epw // B
  mesh = plsc.VectorSubcoreMesh(core_axis_name="c", subcore_axis_name="s")

  @functools.partial(
      pl.kernel,
      out_type=jax.ShapeDtypeStruct((NCORE, NP, 16), jnp.float32),
      mesh=mesh,
      scratch_types=[
          pltpu.VMEM((B,), jnp.int32),
          pltpu.VMEM((B,), jnp.int32),
          pltpu.VMEM((B, 16), jnp.float32),
          pltpu.VMEM((1280, 16), jnp.float32),
          pltpu.SemaphoreType.DMA,
      ],
  )
  def agg_kernel(src_hbm, dst_hbm, z_hbm, out_hbm, sidx, didx, rows, zbuf, sem):
    c = lax.axis_index("c")
    s = lax.axis_index("s")
    wid = c * NSUB + s
    _fill_rows_f32(zbuf, 1280, 0.0)

    def run(acc):
      for t in range(5):
        pltpu.sync_copy(
            zbuf, acc.at[pl.ds(s * ROWS_PER_SUB + t * 1280, 1280)])
      plsc.subcore_barrier()
      base = wid * epw

      def body(i, _):
        off = base + i * B
        pltpu.sync_copy(src_hbm.at[pl.ds(off, B)], sidx)
        pltpu.sync_copy(dst_hbm.at[pl.ds(off, B)], didx)
        pltpu.async_copy(z_hbm.at[sidx], rows, sem).wait()
        pltpu.sync_copy(rows, acc.at[didx], add=True)
        return 0

      lax.fori_loop(0, nblk, body, 0)
      plsc.subcore_barrier()
      sl = pl.ds(s * ROWS_PER_SUB, ROWS_PER_SUB)
      pltpu.sync_copy(acc.at[sl], out_hbm.at[c].at[sl])

    pl.run_scoped(run, pltpu.VMEM_SHARED((NP, 16), jnp.float32))

  return agg_kernel(src, dst, z)


def _sc_agg64_col_split(src, dst, z2c):
  B = 5000
  eps = EE // NSUB
  nblk = eps // B
  mesh = plsc.VectorSubcoreMesh(core_axis_name="c", subcore_axis_name="s")

  @functools.partial(
      pl.kernel,
      out_type=jax.ShapeDtypeStruct((4, NP, 16), jnp.float32),
      mesh=mesh,
      scratch_types=[
          pltpu.VMEM((B,), jnp.int32),
          pltpu.VMEM((B,), jnp.int32),
          pltpu.VMEM((B, 16), jnp.float32),
          pltpu.VMEM((1280, 16), jnp.float32),
          pltpu.SemaphoreType.DMA,
      ],
  )
  def agg2_kernel(src_hbm, dst_hbm, z_hbm, out_hbm, sidx, didx, rows, zbuf,
                  sem):
    c = lax.axis_index("c")
    s = lax.axis_index("s")
    _fill_rows_f32(zbuf, 1280, 0.0)

    def run(acc):
      def run_chunk(k):
        for t in range(5):
          pltpu.sync_copy(
              zbuf, acc.at[pl.ds(s * ROWS_PER_SUB + t * 1280, 1280)])
        plsc.subcore_barrier()
        base = s * eps

        def body(i, _):
          off = base + i * B
          pltpu.sync_copy(src_hbm.at[pl.ds(off, B)], sidx)
          pltpu.sync_copy(dst_hbm.at[pl.ds(off, B)], didx)
          pltpu.async_copy(z_hbm.at[k].at[sidx], rows, sem).wait()
          pltpu.sync_copy(rows, acc.at[didx], add=True)
          return 0

        lax.fori_loop(0, nblk, body, 0)
        plsc.subcore_barrier()
        sl = pl.ds(s * ROWS_PER_SUB, ROWS_PER_SUB)
        pltpu.sync_copy(acc.at[sl], out_hbm.at[k].at[sl])
        plsc.subcore_barrier()

      @pl.when(c == 0)
      def _():
        run_chunk(0)
        run_chunk(1)

      @pl.when(c == 1)
      def _():
        run_chunk(2)
        run_chunk(3)

    pl.run_scoped(run, pltpu.VMEM_SHARED((NP, 16), jnp.float32))

  return agg2_kernel(src, dst, z2c)


def _tc_prep(dega, degb, xp):
  grid = NP // RB

  def body(dega_ref, degb_ref, x_ref, dinv_ref, z1_ref):
    deg = dega_ref[...] + degb_ref[...] + 1.0
    dinv = lax.rsqrt(deg)
    dinv_ref[...] = dinv
    dcol = dinv.reshape(RB, 1)
    z1 = x_ref[...] * dcol
    z1_ref[...] = jnp.concatenate(
        [z1, jnp.zeros((RB, 8), jnp.float32)], axis=1)

  return pl.pallas_call(
      body,
      grid=(grid,),
      in_specs=[
          pl.BlockSpec((RB // 128, 128), lambda i: (i, 0)),
          pl.BlockSpec((RB // 128, 128), lambda i: (i, 0)),
          pl.BlockSpec((RB, 8), lambda i: (i, 0)),
      ],
      out_specs=[
          pl.BlockSpec((RB // 128, 128), lambda i: (i, 0)),
          pl.BlockSpec((RB, 16), lambda i: (i, 0)),
      ],
      out_shape=[
          jax.ShapeDtypeStruct((NP // 128, 128), jnp.float32),
          jax.ShapeDtypeStruct((NP, 16), jnp.float32),
      ],
  )(dega, degb, xp)


def _tc_layer1(agg1p, z1, dinvp, W1, b1):
  grid = NP // RB

  def body(agg_ref, z1_ref, dinv_ref, w_ref, b_ref, out_ref):
    dcol = dinv_ref[...].reshape(RB, 1)
    a1 = (agg_ref[0] + agg_ref[1] + z1_ref[...]) * dcol
    h = jnp.dot(a1[:, :8], w_ref[...], preferred_element_type=jnp.float32)
    h = jnp.maximum(h + b_ref[...], 0.0)
    z2 = h * dcol
    for k in range(4):
      out_ref[k] = z2[:, k * 16:(k + 1) * 16]

  return pl.pallas_call(
      body,
      grid=(grid,),
      in_specs=[
          pl.BlockSpec((2, RB, 16), lambda i: (0, i, 0)),
          pl.BlockSpec((RB, 16), lambda i: (i, 0)),
          pl.BlockSpec((RB // 128, 128), lambda i: (i, 0)),
          pl.BlockSpec((8, 64), lambda i: (0, 0)),
          pl.BlockSpec((1, 64), lambda i: (0, 0)),
      ],
      out_specs=pl.BlockSpec((4, RB, 16), lambda i: (0, i, 0)),
      out_shape=jax.ShapeDtypeStruct((4, NP, 16), jnp.float32),
  )(agg1p, z1, dinvp, W1, b1)


def _tc_final(agg2, z2c, dinvp, W2, b2, Wfc, bfc):
  grid = NP // RB

  def body(agg_ref, z2_ref, dinv_ref, w_ref, b_ref, wfc_ref, bfc_ref,
           out_ref, acc_ref):
    i = pl.program_id(0)

    @pl.when(i == 0)
    def _():
      acc_ref[...] = jnp.zeros_like(acc_ref)

    dcol = dinv_ref[...].reshape(RB, 1)
    a2 = jnp.concatenate(
        [(agg_ref[k] + z2_ref[k]) * dcol for k in range(4)], axis=1)
    h2 = jnp.dot(a2, w_ref[...], preferred_element_type=jnp.float32)
    h2 = jnp.maximum(h2 + b_ref[...], 0.0)
    rowid = i * RB + lax.broadcasted_iota(jnp.int32, (RB, 1), 0)
    h2 = jnp.where(rowid < NN, h2, 0.0)
    acc_ref[...] += jnp.sum(h2, axis=0, keepdims=True)

    @pl.when(i == grid - 1)
    def _():
      g = acc_ref[...] / jnp.float32(NN)
      out_ref[...] = jnp.dot(
          g, wfc_ref[...], preferred_element_type=jnp.float32) + bfc_ref[...]

  return pl.pallas_call(
      body,
      grid=(grid,),
      in_specs=[
          pl.BlockSpec((4, RB, 16), lambda i: (0, i, 0)),
          pl.BlockSpec((4, RB, 16), lambda i: (0, i, 0)),
          pl.BlockSpec((RB // 128, 128), lambda i: (i, 0)),
          pl.BlockSpec((64, 128), lambda i: (0, 0)),
          pl.BlockSpec((1, 128), lambda i: (0, 0)),
          pl.BlockSpec((128, 1), lambda i: (0, 0)),
          pl.BlockSpec((1, 1), lambda i: (0, 0)),
      ],
      out_specs=pl.BlockSpec((1, 1), lambda i: (0, 0)),
      out_shape=jax.ShapeDtypeStruct((1, 1), jnp.float32),
      scratch_shapes=[pltpu.VMEM((1, 128), jnp.float32)],
  )(agg2, z2c, dinvp, W2, b2, Wfc, bfc)


def kernel(x, edge_index, W1, b1, W2, b2, Wfc, bfc):
  src = edge_index[0]
  dst = edge_index[1]
  xp = jnp.pad(x, ((0, NP - NN), (0, 0)))

  degp = _sc_degree(dst)
  dega = degp[0].reshape(NP // 128, 128)
  degb = degp[1].reshape(NP // 128, 128)
  dinvp, z1 = _tc_prep(dega, degb, xp)
  agg1p = _sc_agg16_edge_split(src, dst, z1)
  z2c = _tc_layer1(agg1p, z1, dinvp, W1.astype(jnp.float32),
                   b1.reshape(1, 64))
  agg2 = _sc_agg64_col_split(src, dst, z2c)
  out = _tc_final(agg2, z2c, dinvp, W2.astype(jnp.float32),
                  b2.reshape(1, 128), Wfc, bfc.reshape(1, 1))
  return out.reshape((1,))


# speedup vs baseline: 13.4776x; 13.4776x over previous
import functools

import jax
import jax.numpy as jnp
from jax import lax
from jax.experimental import pallas as pl
from jax.experimental.pallas import tpu as pltpu
from jax.experimental.pallas import tpu_sc as plsc

NN = 100000
EE = 1600000
NP = 102400
TR = 32768
TRBITS = 15
NRANGE = 4
NSUB = 16
NCORE = 2
ROWS_PER_SUB = NP // NSUB
TR_SUB = TR // NSUB
LAST_ROWS = NP - 3 * TR
RB = 6400


def _fill_f32(ref, n, value):
  def body(i, _):
    ref[pl.ds(i * 16, 16)] = jnp.full((16,), value, jnp.float32)
    return 0
  lax.fori_loop(0, n // 16, body, 0)


def _fill_rows_f32(ref, rows, value):
  def body(i, _):
    ref[i, :] = jnp.full((16,), value, jnp.float32)
    return 0
  lax.fori_loop(0, rows, body, 0)


def _sc_degree(dst):
  B = 10000
  epw = EE // (NCORE * NSUB)
  nblk = epw // B
  mesh = plsc.VectorSubcoreMesh(core_axis_name="c", subcore_axis_name="s")

  @functools.partial(
      pl.kernel,
      compiler_params=pltpu.CompilerParams(use_tc_tiling_on_sc=False),
      out_type=jax.ShapeDtypeStruct((NCORE, NP), jnp.float32),
      mesh=mesh,
      scratch_types=[
          pltpu.VMEM((B,), jnp.int32),
          pltpu.VMEM((B,), jnp.float32),
          pltpu.VMEM((ROWS_PER_SUB,), jnp.float32),
          pltpu.VMEM_SHARED((NP,), jnp.float32),
      ],
  )
  def deg_kernel(dst_hbm, out_hbm, idx_v, ones_v, zero_v, acc):
    c = lax.axis_index("c")
    s = lax.axis_index("s")
    wid = c * NSUB + s
    _fill_f32(ones_v, B, 1.0)
    _fill_f32(zero_v, ROWS_PER_SUB, 0.0)

    pltpu.sync_copy(zero_v, acc.at[pl.ds(s * ROWS_PER_SUB, ROWS_PER_SUB)])
    plsc.subcore_barrier()
    base = wid * epw

    def body(i, _):
      pltpu.sync_copy(dst_hbm.at[pl.ds(base + i * B, B)], idx_v)
      pltpu.sync_copy(ones_v, acc.at[idx_v], add=True)
      return 0

    lax.fori_loop(0, nblk, body, 0)
    plsc.subcore_barrier()
    sl = pl.ds(s * ROWS_PER_SUB, ROWS_PER_SUB)
    pltpu.sync_copy(acc.at[sl], out_hbm.at[c].at[sl])

  return deg_kernel(dst)


def _tc_edge_filter(src2d, dst2d):
  EB = 12800
  EC = 125
  BLK = 1600

  def body(s_ref, d_ref, sf_ref, df_ref):
    sv = s_ref[...]
    dv = d_ref[...]
    rng = lax.shift_right_logical(dv, TRBITS)
    dadj = lax.bitwise_and(dv, TR - 1)
    for r in range(NRANGE):
      ok = rng == r
      sf_ref[r] = jnp.where(ok, sv, -1)
      df_ref[r] = jnp.where(ok, dadj, -1)

  return pl.pallas_call(
      body,
      grid=(EB // BLK,),
      in_specs=[
          pl.BlockSpec((BLK, EC), lambda i: (i, 0)),
          pl.BlockSpec((BLK, EC), lambda i: (i, 0)),
      ],
      out_specs=[
          pl.BlockSpec((NRANGE, BLK, EC), lambda i: (0, i, 0)),
          pl.BlockSpec((NRANGE, BLK, EC), lambda i: (0, i, 0)),
      ],
      out_shape=[
          jax.ShapeDtypeStruct((NRANGE, EB, EC), jnp.int32),
          jax.ShapeDtypeStruct((NRANGE, EB, EC), jnp.int32),
      ],
  )(src2d, dst2d)


@functools.cache
def _agg16_kernel():
  B = 5000
  epw = EE // (NCORE * NSUB)
  nblk = epw // B
  mesh = plsc.VectorSubcoreMesh(core_axis_name="c", subcore_axis_name="s")

  @functools.partial(
      pl.kernel,
      compiler_params=pltpu.CompilerParams(use_tc_tiling_on_sc=False),
      out_type=jax.ShapeDtypeStruct((NCORE, NP, 16), jnp.float32),
      mesh=mesh,
      scratch_types=[
          pltpu.VMEM((B,), jnp.int32),
          pltpu.VMEM((B,), jnp.int32),
          pltpu.VMEM((B, 16), jnp.float32),
          pltpu.VMEM((512, 16), jnp.float32),
          pltpu.VMEM_SHARED((TR, 16), jnp.float32),
          pltpu.SemaphoreType.DMA,
      ],
  )
  def agg_kernel(sf_hbm, df_hbm, z_hbm, out_hbm, sidx, didx, rows, zbuf,
                 acc, sem):
    c = lax.axis_index("c")
    s = lax.axis_index("s")
    wid = c * NSUB + s
    _fill_rows_f32(zbuf, 512, 0.0)

    for r in range(NRANGE):
      for t in range(4):
        pltpu.sync_copy(zbuf, acc.at[pl.ds(s * TR_SUB + t * 512, 512)])
      plsc.subcore_barrier()
      base = wid * epw

      def body(i, _):
        off = base + i * B
        pltpu.sync_copy(sf_hbm.at[r].at[pl.ds(off, B)], sidx)
        pltpu.sync_copy(df_hbm.at[r].at[pl.ds(off, B)], didx)
        pltpu.async_copy(
            z_hbm.at[plsc.Indices(sidx, ignored_value=-1)], rows, sem
        ).wait()
        pltpu.sync_copy(
            rows, acc.at[plsc.Indices(didx, ignored_value=-1)], add=True)
        return 0

      lax.fori_loop(0, nblk, body, 0)
      plsc.subcore_barrier()
      nrows = TR_SUB if r < NRANGE - 1 else LAST_ROWS // NSUB
      pltpu.sync_copy(
          acc.at[pl.ds(s * nrows, nrows)],
          out_hbm.at[c].at[pl.ds(r * TR + s * nrows, nrows)])
      plsc.subcore_barrier()

  return agg_kernel


def _sc_agg16(sf, df, z):
  return _agg16_kernel()(sf, df, z)


def _tc_prep(degp2, xp):
  grid = NP // RB

  def body(deg_ref, x_ref, z1_ref):
    dcol = lax.rsqrt(deg_ref[0] + deg_ref[1] + 1.0)
    z1 = x_ref[...] * dcol
    z1_ref[...] = jnp.concatenate(
        [z1, jnp.zeros((RB, 8), jnp.float32)], axis=1)

  return pl.pallas_call(
      body,
      grid=(grid,),
      in_specs=[
          pl.BlockSpec((2, RB, 1), lambda i: (0, i, 0)),
          pl.BlockSpec((RB, 8), lambda i: (i, 0)),
      ],
      out_specs=pl.BlockSpec((RB, 16), lambda i: (i, 0)),
      out_shape=jax.ShapeDtypeStruct((NP, 16), jnp.float32),
  )(degp2, xp)


def _tc_layer1(agg1p, z1, degp2, W1, b1):
  grid = NP // RB

  def body(agg_ref, z1_ref, deg_ref, w_ref, b_ref, out_ref):
    dcol = lax.rsqrt(deg_ref[0] + deg_ref[1] + 1.0)
    a1 = (agg_ref[0] + agg_ref[1] + z1_ref[...]) * dcol
    h = jnp.dot(a1[:, :8], w_ref[...], preferred_element_type=jnp.float32)
    h = jnp.maximum(h + b_ref[...], 0.0)
    out_ref[...] = h * dcol

  return pl.pallas_call(
      body,
      grid=(grid,),
      in_specs=[
          pl.BlockSpec((2, RB, 16), lambda i: (0, i, 0)),
          pl.BlockSpec((RB, 16), lambda i: (i, 0)),
          pl.BlockSpec((2, RB, 1), lambda i: (0, i, 0)),
          pl.BlockSpec((8, 64), lambda i: (0, 0)),
          pl.BlockSpec((1, 64), lambda i: (0, 0)),
      ],
      out_specs=pl.BlockSpec((RB, 64), lambda i: (i, 0)),
      out_shape=jax.ShapeDtypeStruct((NP, 64), jnp.float32),
  )(agg1p, z1, degp2, W1, b1)


def _tc_final(agg2cat, z2, degp2, W2, b2, Wfc, bfc):
  grid = NP // RB

  def body(agg_ref, z2_ref, deg_ref, w_ref, b_ref, wfc_ref, bfc_ref,
           out_ref, acc_ref):
    i = pl.program_id(0)

    @pl.when(i == 0)
    def _():
      acc_ref[...] = jnp.zeros_like(acc_ref)

    dcol = lax.rsqrt(deg_ref[0] + deg_ref[1] + 1.0)
    a2 = (agg_ref[0] + agg_ref[1] + z2_ref[...]) * dcol
    h2 = jnp.dot(a2, w_ref[...], preferred_element_type=jnp.float32)
    h2 = jnp.maximum(h2 + b_ref[...], 0.0)
    rowid = i * RB + lax.broadcasted_iota(jnp.int32, (RB, 1), 0)
    h2 = jnp.where(rowid < NN, h2, 0.0)
    acc_ref[...] += jnp.sum(h2, axis=0, keepdims=True)

    @pl.when(i == grid - 1)
    def _():
      g = acc_ref[...] / jnp.float32(NN)
      out_ref[...] = jnp.dot(
          g, wfc_ref[...], preferred_element_type=jnp.float32) + bfc_ref[...]

  return pl.pallas_call(
      body,
      grid=(grid,),
      in_specs=[
          pl.BlockSpec((2, RB, 64), lambda i: (0, i, 0)),
          pl.BlockSpec((RB, 64), lambda i: (i, 0)),
          pl.BlockSpec((2, RB, 1), lambda i: (0, i, 0)),
          pl.BlockSpec((64, 128), lambda i: (0, 0)),
          pl.BlockSpec((1, 128), lambda i: (0, 0)),
          pl.BlockSpec((128, 1), lambda i: (0, 0)),
          pl.BlockSpec((1, 1), lambda i: (0, 0)),
      ],
      out_specs=pl.BlockSpec((1, 1), lambda i: (0, 0)),
      out_shape=jax.ShapeDtypeStruct((1, 1), jnp.float32),
      scratch_shapes=[pltpu.VMEM((1, 128), jnp.float32)],
  )(agg2cat, z2, degp2, W2, b2, Wfc, bfc)


def kernel(x, edge_index, W1, b1, W2, b2, Wfc, bfc):
  src = edge_index[0]
  dst = edge_index[1]
  sf, df = _tc_edge_filter(src.reshape(12800, 125),
                           dst.reshape(12800, 125))
  sf = sf.reshape(NRANGE, EE)
  df = df.reshape(NRANGE, EE)
  xp = jnp.pad(x, ((0, NP - NN), (0, 0)))

  degp = _sc_degree(dst)
  degp2 = degp.reshape(2, NP, 1)
  z1 = _tc_prep(degp2, xp)
  agg1p = _sc_agg16(sf, df, z1)
  z2 = _tc_layer1(agg1p, z1, degp2, W1.astype(jnp.float32),
                  b1.reshape(1, 64))
  z2c = [z2[:, 16 * k:16 * (k + 1)] for k in range(4)]
  agg2p = [_sc_agg16(sf, df, zc) for zc in z2c]
  agg2cat = jnp.concatenate(agg2p, axis=2)
  out = _tc_final(agg2cat, z2, degp2, W2.astype(jnp.float32),
                  b2.reshape(1, 128), Wfc, bfc.reshape(1, 1))
  return out.reshape((1,))


# speedup vs baseline: 13.6245x; 1.0109x over previous
import functools

import jax
import jax.numpy as jnp
from jax import lax
from jax.experimental import pallas as pl
from jax.experimental.pallas import tpu as pltpu
from jax.experimental.pallas import tpu_sc as plsc

NN = 100000
EE = 1600000
NP = 102400
TR = 32768
TRBITS = 15
NRANGE = 4
NSUB = 16
NCORE = 2
ROWS_PER_SUB = NP // NSUB
TR_SUB = TR // NSUB
LAST_ROWS = NP - 3 * TR
RB = 6400


def _fill_f32(ref, n, value):
  def body(i, _):
    ref[pl.ds(i * 16, 16)] = jnp.full((16,), value, jnp.float32)
    return 0
  lax.fori_loop(0, n // 16, body, 0)


def _fill_rows_f32(ref, rows, value):
  def body(i, _):
    ref[i, :] = jnp.full((16,), value, jnp.float32)
    return 0
  lax.fori_loop(0, rows, body, 0)


def _sc_degree(dst):
  B = 10000
  epw = EE // (NCORE * NSUB)
  nblk = epw // B
  mesh = plsc.VectorSubcoreMesh(core_axis_name="c", subcore_axis_name="s")

  @functools.partial(
      pl.kernel,
      compiler_params=pltpu.CompilerParams(use_tc_tiling_on_sc=False),
      out_type=jax.ShapeDtypeStruct((NCORE, NP), jnp.float32),
      mesh=mesh,
      scratch_types=[
          pltpu.VMEM((B,), jnp.int32),
          pltpu.VMEM((B,), jnp.float32),
          pltpu.VMEM((ROWS_PER_SUB,), jnp.float32),
          pltpu.VMEM_SHARED((NP,), jnp.float32),
      ],
  )
  def deg_kernel(dst_hbm, out_hbm, idx_v, ones_v, zero_v, acc):
    c = lax.axis_index("c")
    s = lax.axis_index("s")
    wid = c * NSUB + s
    _fill_f32(ones_v, B, 1.0)
    _fill_f32(zero_v, ROWS_PER_SUB, 0.0)

    pltpu.sync_copy(zero_v, acc.at[pl.ds(s * ROWS_PER_SUB, ROWS_PER_SUB)])
    plsc.subcore_barrier()
    base = wid * epw

    def body(i, _):
      pltpu.sync_copy(dst_hbm.at[pl.ds(base + i * B, B)], idx_v)
      pltpu.sync_copy(ones_v, acc.at[idx_v], add=True)
      return 0

    lax.fori_loop(0, nblk, body, 0)
    plsc.subcore_barrier()
    sl = pl.ds(s * ROWS_PER_SUB, ROWS_PER_SUB)
    pltpu.sync_copy(acc.at[sl], out_hbm.at[c].at[sl])

  return deg_kernel(dst)


def _tc_edge_filter(src2d, dst2d):
  EB = 12800
  EC = 125
  BLK = 1600

  def body(s_ref, d_ref, sf_ref, df_ref):
    sv = s_ref[...]
    dv = d_ref[...]
    rng = lax.shift_right_logical(dv, TRBITS)
    dadj = lax.bitwise_and(dv, TR - 1)
    for r in range(NRANGE):
      ok = rng == r
      sf_ref[r] = jnp.where(ok, sv, -1)
      df_ref[r] = jnp.where(ok, dadj, -1)

  return pl.pallas_call(
      body,
      grid=(EB // BLK,),
      in_specs=[
          pl.BlockSpec((BLK, EC), lambda i: (i, 0)),
          pl.BlockSpec((BLK, EC), lambda i: (i, 0)),
      ],
      out_specs=[
          pl.BlockSpec((NRANGE, BLK, EC), lambda i: (0, i, 0)),
          pl.BlockSpec((NRANGE, BLK, EC), lambda i: (0, i, 0)),
      ],
      out_shape=[
          jax.ShapeDtypeStruct((NRANGE, EB, EC), jnp.int32),
          jax.ShapeDtypeStruct((NRANGE, EB, EC), jnp.int32),
      ],
  )(src2d, dst2d)


@functools.cache
def _agg16_kernel():
  B = 2000
  epw = EE // (NCORE * NSUB)
  nblk = epw // B
  mesh = plsc.VectorSubcoreMesh(core_axis_name="c", subcore_axis_name="s")

  @functools.partial(
      pl.kernel,
      compiler_params=pltpu.CompilerParams(use_tc_tiling_on_sc=False),
      out_type=jax.ShapeDtypeStruct((NCORE, NP, 16), jnp.float32),
      mesh=mesh,
      scratch_types=[
          pltpu.VMEM((B,), jnp.int32),
          pltpu.VMEM((B,), jnp.int32),
          pltpu.VMEM((B,), jnp.int32),
          pltpu.VMEM((B,), jnp.int32),
          pltpu.VMEM((B, 16), jnp.float32),
          pltpu.VMEM((B, 16), jnp.float32),
          pltpu.VMEM((512, 16), jnp.float32),
          pltpu.VMEM_SHARED((TR, 16), jnp.float32),
          pltpu.SemaphoreType.DMA,
          pltpu.SemaphoreType.DMA,
          pltpu.SemaphoreType.DMA,
      ],
  )
  def agg_kernel(sf_hbm, df_hbm, z_hbm, out_hbm, sidx0, didx0, sidx1, didx1,
                 rows0, rows1, zbuf, acc, semg, sems0, sems1):
    c = lax.axis_index("c")
    s = lax.axis_index("s")
    wid = c * NSUB + s
    _fill_rows_f32(zbuf, 512, 0.0)

    def scat_desc(rows_v, didx_v, sem):
      return pltpu.make_async_copy(
          rows_v, acc.at[plsc.Indices(didx_v, ignored_value=-1)], sem)

    for r in range(NRANGE):
      for t in range(4):
        pltpu.sync_copy(zbuf, acc.at[pl.ds(s * TR_SUB + t * 512, 512)])
      plsc.subcore_barrier()
      base = wid * epw

      def chain(sidx_v, didx_v, rows_v, sem_s, i):
        @pl.when(i >= 2)
        def _():
          scat_desc(rows_v, didx_v, sem_s).wait()

        off = base + i * B
        pltpu.sync_copy(sf_hbm.at[r].at[pl.ds(off, B)], sidx_v)
        pltpu.sync_copy(df_hbm.at[r].at[pl.ds(off, B)], didx_v)
        pltpu.async_copy(
            z_hbm.at[plsc.Indices(sidx_v, ignored_value=-1)], rows_v, semg
        ).wait()
        scat_desc(rows_v, didx_v, sem_s).start(add=True)

      def body(i, _):
        @pl.when(i % 2 == 0)
        def _():
          chain(sidx0, didx0, rows0, sems0, i)

        @pl.when(i % 2 == 1)
        def _():
          chain(sidx1, didx1, rows1, sems1, i)

        return 0

      lax.fori_loop(0, nblk, body, 0)
      scat_desc(rows0, didx0, sems0).wait()
      scat_desc(rows1, didx1, sems1).wait()
      plsc.subcore_barrier()
      nrows = TR_SUB if r < NRANGE - 1 else LAST_ROWS // NSUB
      pltpu.sync_copy(
          acc.at[pl.ds(s * nrows, nrows)],
          out_hbm.at[c].at[pl.ds(r * TR + s * nrows, nrows)])
      plsc.subcore_barrier()

  return agg_kernel


def _sc_agg16(sf, df, z):
  return _agg16_kernel()(sf, df, z)


def _tc_prep(degp2, xp):
  grid = NP // RB

  def body(deg_ref, x_ref, z1_ref):
    dcol = lax.rsqrt(deg_ref[0] + deg_ref[1] + 1.0)
    z1 = x_ref[...] * dcol
    z1_ref[...] = jnp.concatenate(
        [z1, jnp.zeros((RB, 8), jnp.float32)], axis=1)

  return pl.pallas_call(
      body,
      grid=(grid,),
      in_specs=[
          pl.BlockSpec((2, RB, 1), lambda i: (0, i, 0)),
          pl.BlockSpec((RB, 8), lambda i: (i, 0)),
      ],
      out_specs=pl.BlockSpec((RB, 16), lambda i: (i, 0)),
      out_shape=jax.ShapeDtypeStruct((NP, 16), jnp.float32),
  )(degp2, xp)


def _tc_layer1(agg1p, z1, degp2, W1, b1):
  grid = NP // RB

  def body(agg_ref, z1_ref, deg_ref, w_ref, b_ref, out_ref):
    dcol = lax.rsqrt(deg_ref[0] + deg_ref[1] + 1.0)
    a1 = (agg_ref[0] + agg_ref[1] + z1_ref[...]) * dcol
    h = jnp.dot(a1[:, :8], w_ref[...], preferred_element_type=jnp.float32)
    h = jnp.maximum(h + b_ref[...], 0.0)
    out_ref[...] = h * dcol

  return pl.pallas_call(
      body,
      grid=(grid,),
      in_specs=[
          pl.BlockSpec((2, RB, 16), lambda i: (0, i, 0)),
          pl.BlockSpec((RB, 16), lambda i: (i, 0)),
          pl.BlockSpec((2, RB, 1), lambda i: (0, i, 0)),
          pl.BlockSpec((8, 64), lambda i: (0, 0)),
          pl.BlockSpec((1, 64), lambda i: (0, 0)),
      ],
      out_specs=pl.BlockSpec((RB, 64), lambda i: (i, 0)),
      out_shape=jax.ShapeDtypeStruct((NP, 64), jnp.float32),
  )(agg1p, z1, degp2, W1, b1)


def _tc_final(agg2cat, z2, degp2, W2, b2, Wfc, bfc):
  grid = NP // RB

  def body(agg_ref, z2_ref, deg_ref, w_ref, b_ref, wfc_ref, bfc_ref,
           out_ref, acc_ref):
    i = pl.program_id(0)

    @pl.when(i == 0)
    def _():
      acc_ref[...] = jnp.zeros_like(acc_ref)

    dcol = lax.rsqrt(deg_ref[0] + deg_ref[1] + 1.0)
    a2 = (agg_ref[0] + agg_ref[1] + z2_ref[...]) * dcol
    h2 = jnp.dot(a2, w_ref[...], preferred_element_type=jnp.float32)
    h2 = jnp.maximum(h2 + b_ref[...], 0.0)
    rowid = i * RB + lax.broadcasted_iota(jnp.int32, (RB, 1), 0)
    h2 = jnp.where(rowid < NN, h2, 0.0)
    acc_ref[...] += jnp.sum(h2, axis=0, keepdims=True)

    @pl.when(i == grid - 1)
    def _():
      g = acc_ref[...] / jnp.float32(NN)
      out_ref[...] = jnp.dot(
          g, wfc_ref[...], preferred_element_type=jnp.float32) + bfc_ref[...]

  return pl.pallas_call(
      body,
      grid=(grid,),
      in_specs=[
          pl.BlockSpec((2, RB, 64), lambda i: (0, i, 0)),
          pl.BlockSpec((RB, 64), lambda i: (i, 0)),
          pl.BlockSpec((2, RB, 1), lambda i: (0, i, 0)),
          pl.BlockSpec((64, 128), lambda i: (0, 0)),
          pl.BlockSpec((1, 128), lambda i: (0, 0)),
          pl.BlockSpec((128, 1), lambda i: (0, 0)),
          pl.BlockSpec((1, 1), lambda i: (0, 0)),
      ],
      out_specs=pl.BlockSpec((1, 1), lambda i: (0, 0)),
      out_shape=jax.ShapeDtypeStruct((1, 1), jnp.float32),
      scratch_shapes=[pltpu.VMEM((1, 128), jnp.float32)],
  )(agg2cat, z2, degp2, W2, b2, Wfc, bfc)


def kernel(x, edge_index, W1, b1, W2, b2, Wfc, bfc):
  src = edge_index[0]
  dst = edge_index[1]
  sf, df = _tc_edge_filter(src.reshape(12800, 125),
                           dst.reshape(12800, 125))
  sf = sf.reshape(NRANGE, EE)
  df = df.reshape(NRANGE, EE)
  xp = jnp.pad(x, ((0, NP - NN), (0, 0)))

  degp = _sc_degree(dst)
  degp2 = degp.reshape(2, NP, 1)
  z1 = _tc_prep(degp2, xp)
  agg1p = _sc_agg16(sf, df, z1)
  z2 = _tc_layer1(agg1p, z1, degp2, W1.astype(jnp.float32),
                  b1.reshape(1, 64))
  z2c = [z2[:, 16 * k:16 * (k + 1)] for k in range(4)]
  agg2p = [_sc_agg16(sf, df, zc) for zc in z2c]
  agg2cat = jnp.concatenate(agg2p, axis=2)
  out = _tc_final(agg2cat, z2, degp2, W2.astype(jnp.float32),
                  b2.reshape(1, 128), Wfc, bfc.reshape(1, 1))
  return out.reshape((1,))
